# baseline (speedup 1.0000x reference)
import functools

import jax
import jax.numpy as jnp
from jax import lax
from jax.experimental import pallas as pl
from jax.experimental.pallas import tpu as pltpu
from jax.experimental.pallas import tpu_sc as plsc

_NUM_CORES = 2
_NUM_SUBCORES = 16
_NW = _NUM_CORES * _NUM_SUBCORES
_L = 16
_BB = 128


@functools.partial(jax.jit, static_argnums=(3, 4))
def _mask_embed_sc(idx_flat, mask_flat, cols, b, f):
    d = len(cols)
    n_blocks = b // _BB
    n_items = f * n_blocks
    per_w = n_items // _NW
    chunk = per_w * _BB

    mesh = plsc.VectorSubcoreMesh(core_axis_name="c", subcore_axis_name="s")

    @functools.partial(
        pl.kernel,
        out_type=jax.ShapeDtypeStruct((f, d // 8, n_blocks, 8, _BB),
                                      jnp.float32),
        mesh=mesh,
        scratch_types=(
            [pltpu.VMEM((chunk,), jnp.int32),
             pltpu.VMEM((chunk,), jnp.float32)]
            + [pltpu.VMEM((chunk,), jnp.float32) for _ in range(16)]
            + [pltpu.SemaphoreType.DMA, pltpu.SemaphoreType.DMA]
        ),
    )
    def body(idx_hbm, mask_hbm, *rest):
        col_hbm = rest[:d]
        out_hbm = rest[d]
        idx_v, mask_v = rest[d + 1], rest[d + 2]
        g_v = rest[d + 3:d + 3 + d]
        sem_g, sem_o = rest[d + 3 + d:]
        wid = lax.axis_index("s") * _NUM_CORES + lax.axis_index("c")
        base = wid * chunk

        pltpu.sync_copy(idx_hbm.at[pl.ds(base, chunk)], idx_v)
        cps = [pltpu.async_copy(mask_hbm.at[idx_v], mask_v, sem_g)]
        for dd in range(d):
            cps.append(pltpu.async_copy(col_hbm[dd].at[idx_v], g_v[dd],
                                        sem_g))
        for cp in cps:
            cp.wait()

        def mul_body(j, carry):
            m = mask_v[pl.ds(j * _L, _L)]
            sig = 1.0 / (1.0 + jnp.exp(-m))
            for dd in range(d):
                g_v[dd][pl.ds(j * _L, _L)] = g_v[dd][pl.ds(j * _L, _L)] * sig
            return carry

        lax.fori_loop(0, chunk // _L, mul_body, 0)

        ops = []
        for k in range(per_w):
            item = wid * per_w + k
            fi = item // n_blocks
            bb = item % n_blocks
            for dd in range(d):
                ops.append(pltpu.async_copy(
                    g_v[dd].at[pl.ds(k * _BB, _BB)],
                    out_hbm.at[fi, dd // 8, bb, dd % 8], sem_o))
        for op in ops:
            op.wait()

    return body(idx_flat, mask_flat, *cols)


def kernel(x, embedding, mask_weight):
    b, f = x.shape
    d = embedding.shape[1]
    idx_flat = x.T.reshape(f * b)
    mask_flat = mask_weight.reshape(-1)
    emb_t2 = lax.optimization_barrier(embedding.T * jnp.float32(2.0))
    cols = tuple(emb_t2[dd] for dd in range(d))
    out5 = _mask_embed_sc(idx_flat, mask_flat, cols, b, f)
    return out5.transpose((2, 4, 0, 1, 3)).reshape(b, f, d)
